# Initial kernel scaffold; baseline (speedup 1.0000x reference)
import jax
import jax.numpy as jnp
from jax.experimental import pallas as pl


def kernel(xyz, feat_2d, camera_poses, camera_intrinsics, time_ids, view_emb, pos_proj_w, pos_proj_b, feat_proj_w, feat_proj_b, qkv_w, qkv_b, attn_out_w, attn_out_b, ln1_g, ln1_b, ff1_w, ff1_b, ff2_w, ff2_b, ln2_g, ln2_b, out_w, out_b):
    raise NotImplementedError("write your pallas kernel here")


# speedup vs baseline: 1.2396x; 1.2396x over previous
import functools

import jax
import jax.numpy as jnp
import numpy as np
from jax import lax
from jax.experimental import pallas as pl
from jax.experimental.pallas import tpu as pltpu
from jax.experimental.pallas import tpu_sc as plsc

_T, _V = 4, 8
_HP, _WP = 64, 64
_C = 128
_HID = 128
_NHEADS = 4
_DH = _HID // _NHEADS
_TOPK = 4
_NCAM = _T * _V

_BP1 = 256
_BP3 = 128

_SC_NC = 2
_SC_NS = 16
_SC_NW = _SC_NC * _SC_NS
_SC_CH = 128


def _bfr(a):
    return a.astype(jnp.bfloat16).astype(jnp.float32)


def _project_topk_kernel(xh_ref, consts_ref, rows_ref, wts_ref, tv_ref):
    xh = xh_ref[...]
    x = _bfr(xh[:, 0:1])
    y = _bfr(xh[:, 1:2])
    zz = _bfr(xh[:, 2:3])
    cst = consts_ref[...]

    def kr(i):
        return cst[i:i + 1, 0:_NCAM]

    xc = ((x * kr(9) + y * kr(10)) + zz * kr(11)) + kr(12)
    yc = ((x * kr(13) + y * kr(14)) + zz * kr(15)) + kr(16)
    zc = ((x * kr(17) + y * kr(18)) + zz * kr(19)) + kr(20)
    xcb = _bfr(xc)
    ycb = _bfr(yc)
    zcb = _bfr(zc)
    up = xcb * kr(0) + ycb * kr(1) + zcb * kr(2)
    vp = xcb * kr(3) + ycb * kr(4) + zcb * kr(5)
    wp = xcb * kr(6) + ycb * kr(7) + zcb * kr(8)
    w_img = cst[21, 0]
    h_img = cst[21, 1]
    wm = jnp.maximum(wp, 1e-6)
    u = up / wm
    v = vp / wm
    z = zc
    in_img = (u >= 0) & (u < w_img) & (v >= 0) & (v < h_img)
    visible = (z > 1e-4) & in_img
    scores = (1.0 / (jnp.maximum(z, 0.1) + 1e-6)) * visible.astype(jnp.float32)

    lane = lax.broadcasted_iota(jnp.int32, (_BP1, _NCAM), 1)
    lane16 = lax.broadcasted_iota(jnp.int32, (_BP1, 16), 1)
    lane4 = lax.broadcasted_iota(jnp.int32, (_BP1, 4), 1)
    ufac = _WP / jnp.maximum(1.0, w_img)
    vfac = _HP / jnp.maximum(1.0, h_img)

    rows16 = jnp.zeros((_BP1, 16), jnp.int32)
    wts16 = jnp.zeros((_BP1, 16), jnp.float32)
    tv4 = jnp.zeros((_BP1, 4), jnp.int32)
    sc = scores
    for k in range(_TOPK):
        m = jnp.max(sc, axis=1, keepdims=True)
        eq = sc == m
        idx = jnp.min(jnp.where(eq, lane, _NCAM + 1), axis=1, keepdims=True)
        sel = lane == idx
        sc = jnp.where(sel, -1.0, sc)
        u_sel = jnp.sum(jnp.where(sel, u, 0.0), axis=1, keepdims=True)
        v_sel = jnp.sum(jnp.where(sel, v, 0.0), axis=1, keepdims=True)
        z_sel = jnp.sum(jnp.where(sel, z, 0.0), axis=1, keepdims=True)
        u_feat = u_sel * ufac
        v_feat = v_sel * vfac
        gx = 2.0 * (u_feat / float(max(1.0, _WP - 1))) - 1.0
        gy = 2.0 * (v_feat / float(max(1.0, _HP - 1))) - 1.0
        ix = (gx + 1.0) / 2.0 * (_WP - 1)
        iy = (gy + 1.0) / 2.0 * (_HP - 1)
        x0 = jnp.floor(ix)
        x1 = x0 + 1.0
        y0 = jnp.floor(iy)
        y1 = y0 + 1.0
        wx1 = ix - x0
        wx0 = 1.0 - wx1
        wy1 = iy - y0
        wy0 = 1.0 - wy1
        zmask = (z_sel > 1e-4).astype(jnp.float32)
        base = idx * (_HP * _WP)
        corners = ((y0, x0, wy0 * wx0), (y0, x1, wy0 * wx1),
                   (y1, x0, wy1 * wx0), (y1, x1, wy1 * wx1))
        for c, (yy, xx, wgt) in enumerate(corners):
            valid = ((xx >= 0) & (xx <= _WP - 1) & (yy >= 0) & (yy <= _HP - 1))
            xi = jnp.clip(xx, 0, _WP - 1).astype(jnp.int32)
            yi = jnp.clip(yy, 0, _HP - 1).astype(jnp.int32)
            row = base + yi * _WP + xi
            wfin = wgt * valid.astype(jnp.float32) * zmask
            col = k * 4 + c
            rows16 = jnp.where(lane16 == col, row, rows16)
            wts16 = jnp.where(lane16 == col, wfin, wts16)
        tv4 = jnp.where(lane4 == k, idx, tv4)
    rows_ref[...] = rows16
    wts_ref[...] = wts16
    tv_ref[...] = tv4


def _project_topk(xh8, consts, m_pad):
    grid = m_pad // _BP1
    return pl.pallas_call(
        _project_topk_kernel,
        grid=(grid,),
        in_specs=[
            pl.BlockSpec((_BP1, 8), lambda i: (i, 0)),
            pl.BlockSpec((32, 128), lambda i: (0, 0)),
        ],
        out_specs=[
            pl.BlockSpec((_BP1, 16), lambda i: (i, 0)),
            pl.BlockSpec((_BP1, 16), lambda i: (i, 0)),
            pl.BlockSpec((_BP1, 4), lambda i: (i, 0)),
        ],
        out_shape=[
            jax.ShapeDtypeStruct((m_pad, 16), jnp.int32),
            jax.ShapeDtypeStruct((m_pad, 16), jnp.float32),
            jax.ShapeDtypeStruct((m_pad, 4), jnp.int32),
        ],
    )(xh8, consts)


def _sc_gather(table, idx):
    n = idx.shape[0]
    per_w = n // _SC_NW
    chunks = per_w // _SC_CH
    mesh = plsc.VectorSubcoreMesh(core_axis_name="c", subcore_axis_name="s")

    @functools.partial(
        pl.kernel,
        mesh=mesh,
        out_type=jax.ShapeDtypeStruct((n, table.shape[1]), jnp.float32),
        scratch_types=[
            pltpu.VMEM((_SC_CH,), jnp.int32),
            pltpu.VMEM((_SC_CH, table.shape[1]), jnp.float32),
            pltpu.SemaphoreType.DMA,
        ],
    )
    def gk(table_hbm, idx_hbm, out_hbm, idx_v, rows_v, sem):
        wid = lax.axis_index("s") * _SC_NC + lax.axis_index("c")
        base = wid * per_w

        def body(i, carry):
            r0 = base + i * _SC_CH
            pltpu.sync_copy(idx_hbm.at[pl.ds(r0, _SC_CH)], idx_v)
            pltpu.async_copy(table_hbm.at[idx_v], rows_v, sem).wait()
            pltpu.sync_copy(rows_v, out_hbm.at[pl.ds(r0, _SC_CH)])
            return carry

        lax.fori_loop(0, chunks, body, 0)

    return gk(table, idx)


def _dense_kernel(c0_ref, c1_ref, c2_ref, c3_ref, w0_ref, w1_ref, w2_ref,
                  w3_ref, tv_ref, timep_ref, freqs_ref, vemb_ref, ppwt_ref,
                  ppb_ref, wff_ref, wfp_ref, fpb_ref, qkvt_ref, qkvb_ref,
                  aot_ref, aob_ref, g1_ref, b1_ref, ff1t_ref, ff1b_ref,
                  ff2t_ref, ff2b_ref, g2_ref, b2_ref, outwt_ref, outb_ref,
                  out_ref):
    S = 4 * _BP3

    def mm(a, b):
        return jnp.dot(a.astype(jnp.bfloat16), b.astype(jnp.bfloat16),
                       preferred_element_type=jnp.float32)

    def mmx(a, b):
        return jnp.dot(a, b, preferred_element_type=jnp.float32,
                       precision=lax.Precision.HIGHEST)

    sampled = (c0_ref[...] * w0_ref[...] + c1_ref[...] * w1_ref[...]
               + c2_ref[...] * w2_ref[...] + c3_ref[...] * w3_ref[...])

    trow = timep_ref[0:1, :]
    lane128 = lax.broadcasted_iota(jnp.int32, (1, 128), 1)
    big = jnp.int32(1 << 30)
    tmin = jnp.min(jnp.where(lane128 < _T, trow, big))
    tmax = jnp.maximum(jnp.max(jnp.where(lane128 < _T, trow, -big)), tmin + 1)
    trow_f = trow.astype(jnp.float32)
    onehot_t4 = (lax.broadcasted_iota(jnp.int32, (_T, 128), 1)
                 == lax.broadcasted_iota(jnp.int32, (_T, 128), 0)).astype(jnp.float32)
    tvals = lax.dot_general(onehot_t4, trow_f, (((1,), (1,)), ((), ())),
                            preferred_element_type=jnp.float32,
                            precision=lax.Precision.HIGHEST)
    tnorm = (tvals - tmin.astype(jnp.float32)) / (tmax.astype(jnp.float32)
                                                  - tmin.astype(jnp.float32))
    freqs = freqs_ref[0:1, 0:16]
    phases = tnorm * freqs
    sinp = jnp.sin(phases)
    cosp = jnp.cos(phases)

    r32 = lax.broadcasted_iota(jnp.int32, (_NCAM, _T), 0)
    ot = (r32 // _V == lax.broadcasted_iota(jnp.int32, (_NCAM, _T), 1)).astype(jnp.float32)
    r32v = lax.broadcasted_iota(jnp.int32, (_NCAM, _V), 0)
    ov = (r32v % _V == lax.broadcasted_iota(jnp.int32, (_NCAM, _V), 1)).astype(jnp.float32)
    vpart = mm(ov, vemb_ref[0:_V, :])
    ppwt = ppwt_ref[...]
    pos32 = (mm(mm(ot, sinp), ppwt[0:16, :])
             + mm(mm(ot, cosp), ppwt[16:32, :])
             + mm(vpart, ppwt[32:48, :])
             + ppb_ref[0:1, :])
    posh32 = mm(pos32, wfp_ref[...])

    tv = tv_ref[...]
    oh_tv = (lax.broadcasted_iota(jnp.int32, (S, _NCAM), 1) == tv).astype(jnp.float32)
    h = mm(sampled, wff_ref[...]) + mmx(oh_tv, posh32) + fpb_ref[0:1, :]

    io0 = lax.broadcasted_iota(jnp.int32, (S, S), 0)
    io1 = lax.broadcasted_iota(jnp.int32, (S, S), 1)
    blockmask = (io0 // 4) == (io1 // 4)
    scale = np.float32(np.sqrt(_DH))

    x = h
    for l in range(2):
        qkv = mm(x, qkvt_ref[l]) + qkvb_ref[l:l + 1, :]
        q = qkv[:, 0:_HID]
        kk = qkv[:, _HID:2 * _HID]
        vv = qkv[:, 2 * _HID:3 * _HID]
        aot = aot_ref[l]
        o = aob_ref[l:l + 1, :]
        for hh in range(_NHEADS):
            sl0 = hh * _DH
            qh = q[:, sl0:sl0 + _DH]
            kh = kk[:, sl0:sl0 + _DH]
            vh = vv[:, sl0:sl0 + _DH]
            s = lax.dot_general(qh.astype(jnp.bfloat16), kh.astype(jnp.bfloat16),
                                (((1,), (1,)), ((), ())),
                                preferred_element_type=jnp.float32) / scale
            s = jnp.where(blockmask, s, -1e30)
            smax = jnp.max(s, axis=1, keepdims=True)
            e = jnp.exp(s - smax)
            p = e / jnp.sum(e, axis=1, keepdims=True)
            o = o + mm(mm(p, vh), aot[sl0:sl0 + _DH, :])
        xo = x + o
        mu = jnp.mean(xo, axis=1, keepdims=True)
        var = jnp.mean((xo - mu) ** 2, axis=1, keepdims=True)
        x = (xo - mu) / jnp.sqrt(var + 1e-5) * g1_ref[l:l + 1, :] + b1_ref[l:l + 1, :]
        f = jnp.maximum(mm(x, ff1t_ref[l]) + ff1b_ref[l:l + 1, :], 0.0)
        f = mm(f, ff2t_ref[l]) + ff2b_ref[l:l + 1, :]
        xf = x + f
        mu = jnp.mean(xf, axis=1, keepdims=True)
        var = jnp.mean((xf - mu) ** 2, axis=1, keepdims=True)
        x = (xf - mu) / jnp.sqrt(var + 1e-5) * g2_ref[l:l + 1, :] + b2_ref[l:l + 1, :]

    mrow = lax.broadcasted_iota(jnp.int32, (_BP3, S), 0)
    mcol = lax.broadcasted_iota(jnp.int32, (_BP3, S), 1)
    mavg = jnp.where(mcol // 4 == mrow, 0.25, 0.0).astype(jnp.float32)
    g = mmx(mavg, x)
    out_ref[...] = mm(g, outwt_ref[...]) + outb_ref[0:1, :]


def _dense(corners, wtsc, tv1, timep, freqsp, vemb, ppwt, ppb, wff, wfp, fpb,
           qkvt, qkvb, aot, aob, g1, b1, ff1t, ff1b, ff2t, ff2b, g2, b2,
           outwt, outb, m_pad):
    grid = m_pad // _BP3
    S = 4 * _BP3
    pair_spec = pl.BlockSpec((S, 128), lambda i: (i, 0))
    w_spec = pl.BlockSpec((S, 1), lambda i: (i, 0))

    def full(shape):
        nd = len(shape)
        return pl.BlockSpec(shape, lambda i, _n=nd: (0,) * _n)

    return pl.pallas_call(
        _dense_kernel,
        grid=(grid,),
        in_specs=[
            pair_spec, pair_spec, pair_spec, pair_spec,
            w_spec, w_spec, w_spec, w_spec,
            pl.BlockSpec((S, 1), lambda i: (i, 0)),
            full(timep.shape), full(freqsp.shape), full(vemb.shape),
            full(ppwt.shape), full(ppb.shape), full(wff.shape),
            full(wfp.shape), full(fpb.shape), full(qkvt.shape),
            full(qkvb.shape), full(aot.shape), full(aob.shape),
            full(g1.shape), full(b1.shape), full(ff1t.shape),
            full(ff1b.shape), full(ff2t.shape), full(ff2b.shape),
            full(g2.shape), full(b2.shape), full(outwt.shape),
            full(outb.shape),
        ],
        out_specs=pl.BlockSpec((_BP3, 128), lambda i: (i, 0)),
        out_shape=jax.ShapeDtypeStruct((m_pad, 128), jnp.float32),
    )(*corners, *wtsc, tv1, timep, freqsp, vemb, ppwt, ppb, wff, wfp, fpb,
      qkvt, qkvb, aot, aob, g1, b1, ff1t, ff1b, ff2t, ff2b, g2, b2,
      outwt, outb)


def _pad8(a):
    return jnp.pad(a, ((0, 8 - a.shape[0]), (0, 0)))


def kernel(xyz, feat_2d, camera_poses, camera_intrinsics, time_ids, view_emb,
           pos_proj_w, pos_proj_b, feat_proj_w, feat_proj_b, qkv_w, qkv_b,
           attn_out_w, attn_out_b, ln1_g, ln1_b, ff1_w, ff1_b, ff2_w, ff2_b,
           ln2_g, ln2_b, out_w, out_b):
    m = xyz.shape[0]
    m_pad = -(-m // 256) * 256
    npairs = m_pad * _TOPK

    poses = camera_poses.reshape(_NCAM, 4, 4).astype(jnp.float32)
    ks = camera_intrinsics.reshape(_NCAM, 3, 3).astype(jnp.float32)
    w2c = jnp.linalg.inv(poses)
    bfr = lambda a: a.astype(jnp.bfloat16).astype(jnp.float32)
    krows = bfr(ks.transpose(1, 2, 0).reshape(9, _NCAM))
    w2crows = bfr(w2c[:, :3, :].transpose(1, 2, 0).reshape(12, _NCAM))
    cx = camera_intrinsics[0, 0, 0, 2]
    cy = camera_intrinsics[0, 0, 1, 2]
    consts = jnp.zeros((32, 128), jnp.float32)
    consts = consts.at[0:9, 0:_NCAM].set(krows)
    consts = consts.at[9:21, 0:_NCAM].set(w2crows)
    consts = consts.at[21, 0].set(2.0 * cx)
    consts = consts.at[21, 1].set(2.0 * cy)

    xh8 = jnp.zeros((m_pad, 8), jnp.float32)
    xh8 = xh8.at[:m, 0:3].set(xyz.astype(jnp.float32))

    rows16, wts16, tv4 = _project_topk(xh8, consts, m_pad)

    idx_cm = rows16.reshape(m_pad, 4, 4).transpose(2, 0, 1).reshape(-1)
    wts_cm = wts16.reshape(m_pad, 4, 4).transpose(2, 0, 1).reshape(4, npairs, 1)

    table = feat_2d.reshape(_NCAM * _HP * _WP, _C).astype(jnp.float32)
    gathered = _sc_gather(table, idx_cm)
    corners = [lax.slice_in_dim(gathered, c * npairs, (c + 1) * npairs)
               for c in range(4)]
    wtsc = [wts_cm[c] for c in range(4)]
    tv1 = tv4.reshape(npairs, 1)

    timep = jnp.zeros((8, 128), jnp.int32).at[0, 0:_T].set(time_ids.astype(jnp.int32))
    half = 16
    freqs = jnp.exp(jnp.linspace(0.0, 8.0, half))
    freqsp = jnp.zeros((8, 128), jnp.float32).at[0, 0:half].set(freqs)

    out = _dense(
        corners, wtsc, tv1, timep, freqsp, view_emb.astype(jnp.float32),
        pos_proj_w.T, _pad8(pos_proj_b.reshape(1, -1)),
        feat_proj_w[:, 0:_C].T, feat_proj_w[:, _C:_C + 32].T,
        _pad8(feat_proj_b.reshape(1, -1)),
        qkv_w.transpose(0, 2, 1), _pad8(qkv_b),
        attn_out_w.transpose(0, 2, 1), _pad8(attn_out_b),
        _pad8(ln1_g), _pad8(ln1_b),
        ff1_w.transpose(0, 2, 1), _pad8(ff1_b),
        ff2_w.transpose(0, 2, 1), _pad8(ff2_b),
        _pad8(ln2_g), _pad8(ln2_b),
        out_w.T, _pad8(out_b.reshape(1, -1)), m_pad)
    return out[:m]
